# baseline (speedup 1.0000x reference)
import functools

import jax
import jax.numpy as jnp
from jax import lax
from jax.experimental import pallas as pl
from jax.experimental.pallas import tpu as pltpu
from jax.experimental.pallas import tpu_sc as plsc

VOCAB = 100000
EMBED = 128
B = 1024
L = 200
N = B * L

VTILE = 10000
VGRID = VOCAB // VTILE

NTILE = 12800
NGRID = N // NTILE

NSPLIT = 2
N_S = N // NSPLIT
NGRID_S = N_S // NTILE
NW = 32
PER_W = N_S // NW
CHUNK = 80
NCHUNK = PER_W // CHUNK


def _contract(x, w):
    return lax.dot_general(x, w, (((1,), (1,)), ((), ())),
                           preferred_element_type=jnp.float32)


def _fold_kernel(common_ref, demo_ref, rep_ref, w_ref, c_ref):
    w1 = w_ref[:, :EMBED]
    w2 = w_ref[:, EMBED:]
    ws = w1 + w2
    c_ref[...] = (0.25 * _contract(common_ref[...], ws)
                  + 0.5 * _contract(demo_ref[...], w1)
                  + 0.5 * _contract(rep_ref[...], w2))


def _final_kernel(word_ref, g_ref, w_ref, b_ref, o_ref):
    ws = w_ref[:, :EMBED] + w_ref[:, EMBED:]
    o_ref[...] = (0.25 * _contract(word_ref[...], ws)
                  + g_ref[...] + b_ref[...])


def _final_kernel_acc(word_ref, g_ref, w_ref, b_ref, acc_ref, o_ref):
    del acc_ref
    _final_kernel(word_ref, g_ref, w_ref, b_ref, o_ref)


def _sc_gather(texts3d, table):
    mesh = plsc.VectorSubcoreMesh(core_axis_name="c", subcore_axis_name="s")

    @functools.partial(
        pl.kernel,
        out_type=jax.ShapeDtypeStruct((N_S, EMBED), jnp.float32),
        mesh=mesh,
        scratch_types=[
            pltpu.VMEM((NCHUNK, CHUNK), jnp.int32),
            pltpu.VMEM((4, CHUNK, EMBED), jnp.float32),
            pltpu.SemaphoreType.DMA,
            pltpu.SemaphoreType.DMA,
            pltpu.SemaphoreType.DMA,
            pltpu.SemaphoreType.DMA,
        ],
    )
    def gather(texts_hbm, table_hbm, out_hbm, idx_v, rows_v,
               sem_a, sem_b, sem_c, sem_d):
        wid = lax.axis_index("s") * 2 + lax.axis_index("c")
        base = wid * PER_W
        pltpu.sync_copy(texts_hbm.at[wid], idx_v)

        buf = (rows_v.at[0], rows_v.at[1], rows_v.at[2], rows_v.at[3])
        sem = (sem_a, sem_b, sem_c, sem_d)

        def start_gather(j, bi):
            pltpu.async_copy(table_hbm.at[idx_v.at[j]], buf[bi], sem[bi])

        def wait(bi):
            pltpu.make_async_copy(table_hbm.at[pl.ds(0, CHUNK), :], buf[bi],
                                  sem[bi]).wait()

        def start_store(j, bi):
            pltpu.async_copy(buf[bi], out_hbm.at[pl.ds(base + j * CHUNK, CHUNK), :],
                             sem[bi])

        for bi in range(4):
            start_gather(bi, bi)

        def body(i, _):
            j = 4 * i
            for bi in range(4):
                wait(bi)
                start_store(j + bi, bi)
            for bi in range(4):
                wait(bi)
                start_gather(j + 4 + bi, bi)
            return 0

        lax.fori_loop(0, NCHUNK // 4 - 1, body, 0)

        j = NCHUNK - 4
        for bi in range(4):
            wait(bi)
            start_store(j + bi, bi)
        for bi in range(4):
            wait(bi)

    return gather(texts3d, table)


def kernel(word_embeddings, texts, common_tbl, demo_tbl, rep_tbl, W, b):
    texts4d = texts.astype(jnp.int32).reshape(NSPLIT, NW, NCHUNK, CHUNK)

    folded = pl.pallas_call(
        _fold_kernel,
        grid=(VGRID,),
        in_specs=[
            pl.BlockSpec((VTILE, EMBED), lambda i: (i, 0)),
            pl.BlockSpec((VTILE, EMBED), lambda i: (i, 0)),
            pl.BlockSpec((VTILE, EMBED), lambda i: (i, 0)),
            pl.BlockSpec((EMBED, 2 * EMBED), lambda i: (0, 0)),
        ],
        out_specs=pl.BlockSpec((VTILE, EMBED), lambda i: (i, 0)),
        out_shape=jax.ShapeDtypeStruct((VOCAB, EMBED), jnp.float32),
    )(common_tbl, demo_tbl, rep_tbl, W)

    gathered = [_sc_gather(texts4d[k], folded) for k in range(NSPLIT)]

    word2d = word_embeddings.reshape(N, EMBED)
    b2d = b.reshape(1, EMBED)
    out2d = None
    for k in range(NSPLIT):
        base = k * NGRID_S
        in_specs = [
            pl.BlockSpec((NTILE, EMBED), lambda i, base=base: (base + i, 0)),
            pl.BlockSpec((NTILE, EMBED), lambda i: (i, 0)),
            pl.BlockSpec((EMBED, 2 * EMBED), lambda i: (0, 0)),
            pl.BlockSpec((1, EMBED), lambda i: (0, 0)),
        ]
        args = [word2d, gathered[k], W, b2d]
        if k == 0:
            body, aliases = _final_kernel, {}
        else:
            body, aliases = _final_kernel_acc, {4: 0}
            in_specs.append(pl.BlockSpec(memory_space=pltpu.MemorySpace.HBM))
            args.append(out2d)
        out2d = pl.pallas_call(
            body,
            grid=(NGRID_S,),
            in_specs=in_specs,
            out_specs=pl.BlockSpec((NTILE, EMBED),
                                   lambda i, base=base: (base + i, 0)),
            out_shape=jax.ShapeDtypeStruct((N, EMBED), jnp.float32),
            input_output_aliases=aliases,
        )(*args)

    return out2d.reshape(B, L, EMBED)


# speedup vs baseline: 1.0012x; 1.0012x over previous
import functools

import jax
import jax.numpy as jnp
from jax import lax
from jax.experimental import pallas as pl
from jax.experimental.pallas import tpu as pltpu
from jax.experimental.pallas import tpu_sc as plsc

VOCAB = 100000
EMBED = 128
B = 1024
L = 200
N = B * L

VTILE = 10000
VGRID = VOCAB // VTILE

NTILE = 12800
NGRID = N // NTILE

NSPLIT = 1
N_S = N // NSPLIT
NGRID_S = N_S // NTILE
NW = 32
PER_W = N_S // NW
CHUNK = 80
NCHUNK = PER_W // CHUNK
NBUF = 8


def _contract(x, w):
    return lax.dot_general(x, w, (((1,), (1,)), ((), ())),
                           preferred_element_type=jnp.float32)


def _fold_kernel(common_ref, demo_ref, rep_ref, w_ref, c_ref):
    w1 = w_ref[:, :EMBED]
    w2 = w_ref[:, EMBED:]
    ws = w1 + w2
    c_ref[...] = (0.25 * _contract(common_ref[...], ws)
                  + 0.5 * _contract(demo_ref[...], w1)
                  + 0.5 * _contract(rep_ref[...], w2))


def _final_kernel(word_ref, g_ref, w_ref, b_ref, o_ref):
    ws = w_ref[:, :EMBED] + w_ref[:, EMBED:]
    o_ref[...] = (0.25 * _contract(word_ref[...], ws)
                  + g_ref[...] + b_ref[...])


def _final_kernel_acc(word_ref, g_ref, w_ref, b_ref, acc_ref, o_ref):
    del acc_ref
    _final_kernel(word_ref, g_ref, w_ref, b_ref, o_ref)


def _sc_gather(texts3d, table):
    mesh = plsc.VectorSubcoreMesh(core_axis_name="c", subcore_axis_name="s")

    @functools.partial(
        pl.kernel,
        out_type=jax.ShapeDtypeStruct((N_S, EMBED), jnp.float32),
        mesh=mesh,
        scratch_types=[
            pltpu.VMEM((NCHUNK, CHUNK), jnp.int32),
            pltpu.VMEM((NBUF, CHUNK, EMBED), jnp.float32),
        ] + [pltpu.SemaphoreType.DMA] * NBUF,
    )
    def gather(texts_hbm, table_hbm, out_hbm, idx_v, rows_v, *sems):
        wid = lax.axis_index("s") * 2 + lax.axis_index("c")
        base = wid * PER_W
        pltpu.sync_copy(texts_hbm.at[wid], idx_v)

        buf = tuple(rows_v.at[i] for i in range(NBUF))
        sem = sems

        def start_gather(j, bi):
            pltpu.async_copy(table_hbm.at[idx_v.at[j]], buf[bi], sem[bi])

        def wait(bi):
            pltpu.make_async_copy(table_hbm.at[pl.ds(0, CHUNK), :], buf[bi],
                                  sem[bi]).wait()

        def start_store(j, bi):
            pltpu.async_copy(buf[bi], out_hbm.at[pl.ds(base + j * CHUNK, CHUNK), :],
                             sem[bi])

        for bi in range(NBUF):
            start_gather(bi, bi)

        def body(i, _):
            j = NBUF * i
            for bi in range(NBUF):
                wait(bi)
                start_store(j + bi, bi)
            for bi in range(NBUF):
                wait(bi)
                start_gather(j + NBUF + bi, bi)
            return 0

        lax.fori_loop(0, NCHUNK // NBUF - 1, body, 0)

        j = NCHUNK - NBUF
        for bi in range(NBUF):
            wait(bi)
            start_store(j + bi, bi)
        for bi in range(NBUF):
            wait(bi)

    return gather(texts3d, table)


def kernel(word_embeddings, texts, common_tbl, demo_tbl, rep_tbl, W, b):
    texts4d = texts.astype(jnp.int32).reshape(NSPLIT, NW, NCHUNK, CHUNK)

    folded = pl.pallas_call(
        _fold_kernel,
        grid=(VGRID,),
        in_specs=[
            pl.BlockSpec((VTILE, EMBED), lambda i: (i, 0)),
            pl.BlockSpec((VTILE, EMBED), lambda i: (i, 0)),
            pl.BlockSpec((VTILE, EMBED), lambda i: (i, 0)),
            pl.BlockSpec((EMBED, 2 * EMBED), lambda i: (0, 0)),
        ],
        out_specs=pl.BlockSpec((VTILE, EMBED), lambda i: (i, 0)),
        out_shape=jax.ShapeDtypeStruct((VOCAB, EMBED), jnp.float32),
    )(common_tbl, demo_tbl, rep_tbl, W)

    gathered = [_sc_gather(texts4d[k], folded) for k in range(NSPLIT)]

    word2d = word_embeddings.reshape(N, EMBED)
    b2d = b.reshape(1, EMBED)
    out2d = None
    for k in range(NSPLIT):
        base = k * NGRID_S
        in_specs = [
            pl.BlockSpec((NTILE, EMBED), lambda i, base=base: (base + i, 0)),
            pl.BlockSpec((NTILE, EMBED), lambda i: (i, 0)),
            pl.BlockSpec((EMBED, 2 * EMBED), lambda i: (0, 0)),
            pl.BlockSpec((1, EMBED), lambda i: (0, 0)),
        ]
        args = [word2d, gathered[k], W, b2d]
        if k == 0:
            body, aliases = _final_kernel, {}
        else:
            body, aliases = _final_kernel_acc, {4: 0}
            in_specs.append(pl.BlockSpec(memory_space=pltpu.MemorySpace.HBM))
            args.append(out2d)
        out2d = pl.pallas_call(
            body,
            grid=(NGRID_S,),
            in_specs=in_specs,
            out_specs=pl.BlockSpec((NTILE, EMBED),
                                   lambda i, base=base: (base + i, 0)),
            out_shape=jax.ShapeDtypeStruct((N, EMBED), jnp.float32),
            input_output_aliases=aliases,
        )(*args)

    return out2d.reshape(B, L, EMBED)
